# baseline (speedup 1.0000x reference)
import jax
import jax.numpy as jnp
from jax.experimental import pallas as pl

BETA = 0.9
ALPHA = 0.2
LOG2E = 1.4426950408889634


def _prologue_body(x_ref, Wi_ref, bi_ref, Wa_ref, a1_ref, a2_ref,
                   wh_ref, f1_ref, f2_ref, whsum_ref, m2_ref):
    p = pl.program_id(0)
    h = jnp.dot(x_ref[...], Wi_ref[...],
                preferred_element_type=jnp.float32) + bi_ref[...]
    wh = jnp.dot(h, Wa_ref[...], preferred_element_type=jnp.float32)
    f1 = jnp.dot(wh, a1_ref[...], preferred_element_type=jnp.float32)
    f2 = jnp.dot(wh, a2_ref[...], preferred_element_type=jnp.float32)
    wh_ref[...] = wh
    f1_ref[...] = f1
    f2_ref[...] = f2

    @pl.when(p == 0)
    def _init():
        whsum_ref[...] = jnp.zeros_like(whsum_ref)
        m2_ref[...] = jnp.full_like(m2_ref, -jnp.inf)

    whsum_ref[...] += jnp.sum(wh, axis=0, keepdims=True)
    m2_ref[...] = jnp.maximum(m2_ref[...], jnp.max(f2))


def _attn_body(n, nhid, k_hops,
               m2_ref, f1_ref, F2_ref, F2p_ref, *refs):
    mask_refs = refs[:k_hops]
    wha_ref, whsum_ref, aW_ref, ab_ref, oW_ref, ob_ref, out_ref = \
        refs[k_hops:]
    unif = whsum_ref[...] * (1.0 / n)

    f1 = f1_ref[...]
    v = f1 + m2_ref[0, 0]
    c = jnp.maximum(v, ALPHA * v)
    U = (f1 - c) * LOG2E
    Up = (ALPHA * f1 - c) * LOG2E
    t = jnp.maximum(U + F2_ref[...], F2p_ref[...] + Up)
    q = jnp.exp2(t)
    hp = None
    for k in range(k_hops):
        p_k = jnp.where(mask_refs[k][...] != 0, q, 0.0)
        a = jnp.dot(p_k, wha_ref[...], preferred_element_type=jnp.float32)
        s = a[:, nhid:]
        term = jnp.where(s > 0.0, a[:, :nhid] / s, unif)
        hp = term if hp is None else hp + (BETA ** k) * term

    g = jnp.where(hp > 0, hp, jnp.exp(hp) - 1.0)
    lin = jnp.dot(g, aW_ref[...],
                  preferred_element_type=jnp.float32) + ab_ref[...]
    h2 = (1.0 - ALPHA) * lin + ALPHA * g
    o = jnp.dot(h2, oW_ref[...],
                preferred_element_type=jnp.float32) + ob_ref[...]
    o = o - jnp.max(o, axis=1, keepdims=True)
    out_ref[...] = o - jnp.log(jnp.sum(jnp.exp(o), axis=1, keepdims=True))


def _pick(n, pref):
    return pref if n % pref == 0 else n


def kernel(x, hop_masks, W_init, b_init, att_W, att_a, appnp_W, appnp_b,
           W_out, b_out):
    n, nfeat = x.shape
    nhid = W_init.shape[1]
    nclass = W_out.shape[1]
    k_hops = hop_masks.shape[0]

    bp = _pick(n, 1000)
    bi = _pick(n, 1000)
    f32 = jnp.float32

    a1 = att_a[:nhid]
    a2 = att_a[nhid:]

    wh, f1, f2, whsum, m2 = pl.pallas_call(
        _prologue_body,
        grid=(n // bp,),
        in_specs=[
            pl.BlockSpec((bp, nfeat), lambda p: (p, 0)),
            pl.BlockSpec((nfeat, nhid), lambda p: (0, 0)),
            pl.BlockSpec((1, nhid), lambda p: (0, 0)),
            pl.BlockSpec((nhid, nhid), lambda p: (0, 0)),
            pl.BlockSpec((nhid, 1), lambda p: (0, 0)),
            pl.BlockSpec((nhid, 1), lambda p: (0, 0)),
        ],
        out_specs=[
            pl.BlockSpec((bp, nhid), lambda p: (p, 0)),
            pl.BlockSpec((bp, 1), lambda p: (p, 0)),
            pl.BlockSpec((bp, 1), lambda p: (p, 0)),
            pl.BlockSpec((1, nhid), lambda p: (0, 0)),
            pl.BlockSpec((1, 1), lambda p: (0, 0)),
        ],
        out_shape=[
            jax.ShapeDtypeStruct((n, nhid), f32),
            jax.ShapeDtypeStruct((n, 1), f32),
            jax.ShapeDtypeStruct((n, 1), f32),
            jax.ShapeDtypeStruct((1, nhid), f32),
            jax.ShapeDtypeStruct((1, 1), f32),
        ],
    )(x, W_init, b_init.reshape(1, nhid), att_W, a1, a2)

    f2row = f2.reshape(1, n)
    F2 = f2row * LOG2E
    F2p = F2 * ALPHA
    wha = jnp.concatenate([wh, jnp.ones((n, 1), f32)], axis=1)

    mask_args = [hop_masks[k].view(jnp.uint8) for k in range(k_hops)]
    mask_specs = [pl.BlockSpec((bi, n), lambda i: (i, 0))
                  for _ in range(k_hops)]

    def attn(*refs):
        _attn_body(n, nhid, k_hops, *refs)

    out = pl.pallas_call(
        attn,
        grid=(n // bi,),
        in_specs=[
            pl.BlockSpec((1, 1), lambda i: (0, 0)),
            pl.BlockSpec((bi, 1), lambda i: (i, 0)),
            pl.BlockSpec((1, n), lambda i: (0, 0)),
            pl.BlockSpec((1, n), lambda i: (0, 0)),
            *mask_specs,
            pl.BlockSpec((n, nhid + 1), lambda i: (0, 0)),
            pl.BlockSpec((1, nhid), lambda i: (0, 0)),
            pl.BlockSpec((nhid, nhid), lambda i: (0, 0)),
            pl.BlockSpec((1, nhid), lambda i: (0, 0)),
            pl.BlockSpec((nhid, nclass), lambda i: (0, 0)),
            pl.BlockSpec((1, nclass), lambda i: (0, 0)),
        ],
        out_specs=pl.BlockSpec((bi, nclass), lambda i: (i, 0)),
        out_shape=jax.ShapeDtypeStruct((n, nclass), f32),
    )(m2, f1, F2, F2p, *mask_args, wha, whsum, appnp_W,
      appnp_b.reshape(1, nhid), W_out, b_out.reshape(1, nclass))
    return out


# speedup vs baseline: 1.0310x; 1.0310x over previous
import jax
import jax.numpy as jnp
from jax.experimental import pallas as pl

BETA = 0.9
ALPHA = 0.2
LOG2E = 1.4426950408889634


def _prologue_body(x_ref, Wi_ref, bi_ref, Wa_ref, a1_ref, a2_ref,
                   wha_ref, f1_ref, f2L_ref, whsum_ref, m2_ref):
    nhid = Wi_ref.shape[1]
    h = jnp.dot(x_ref[...], Wi_ref[...],
                preferred_element_type=jnp.float32) + bi_ref[...]
    wh = jnp.dot(h, Wa_ref[...], preferred_element_type=jnp.float32)
    f1 = jnp.dot(wh, a1_ref[...], preferred_element_type=jnp.float32)
    f2 = jnp.dot(wh, a2_ref[...], preferred_element_type=jnp.float32)
    wha_ref[:, :nhid] = wh
    wha_ref[:, nhid:] = jnp.ones_like(wha_ref[:, nhid:])
    f1_ref[...] = f1
    f2L_ref[...] = f2 * LOG2E
    whsum_ref[...] = jnp.sum(wh, axis=0, keepdims=True)
    m2_ref[...] = jnp.full_like(m2_ref, jnp.max(f2))


def _attn_body(n, nhid, k_hops,
               m2_ref, f1_ref, F2_ref, F2p_ref, *refs):
    mask_refs = refs[:k_hops]
    wha_ref, whsum_ref, aW_ref, ab_ref, oW_ref, ob_ref, out_ref = \
        refs[k_hops:]
    unif = whsum_ref[...] * (1.0 / n)

    f1 = f1_ref[...]
    v = f1 + m2_ref[0, 0]
    c = jnp.maximum(v, ALPHA * v)
    U = (f1 - c) * LOG2E
    Up = (ALPHA * f1 - c) * LOG2E
    t = jnp.maximum(U + F2_ref[...], F2p_ref[...] + Up)
    q = jnp.exp2(t)
    hp = None
    for k in range(k_hops):
        p_k = jnp.where(mask_refs[k][...] != 0, q, 0.0)
        a = jnp.dot(p_k, wha_ref[...], preferred_element_type=jnp.float32)
        s = a[:, nhid:]
        term = jnp.where(s > 0.0, a[:, :nhid] / s, unif)
        hp = term if hp is None else hp + (BETA ** k) * term

    g = jnp.where(hp > 0, hp, jnp.exp(hp) - 1.0)
    lin = jnp.dot(g, aW_ref[...],
                  preferred_element_type=jnp.float32) + ab_ref[...]
    h2 = (1.0 - ALPHA) * lin + ALPHA * g
    o = jnp.dot(h2, oW_ref[...],
                preferred_element_type=jnp.float32) + ob_ref[...]
    o = o - jnp.max(o, axis=1, keepdims=True)
    out_ref[...] = o - jnp.log(jnp.sum(jnp.exp(o), axis=1, keepdims=True))


def _pick(n, pref):
    return pref if n % pref == 0 else n


def kernel(x, hop_masks, W_init, b_init, att_W, att_a, appnp_W, appnp_b,
           W_out, b_out):
    n, nfeat = x.shape
    nhid = W_init.shape[1]
    nclass = W_out.shape[1]
    k_hops = hop_masks.shape[0]

    bi = _pick(n, 1000)
    f32 = jnp.float32

    a1 = att_a[:nhid]
    a2 = att_a[nhid:]

    wha, f1, f2L, whsum, m2 = pl.pallas_call(
        _prologue_body,
        out_shape=[
            jax.ShapeDtypeStruct((n, nhid + 1), f32),
            jax.ShapeDtypeStruct((n, 1), f32),
            jax.ShapeDtypeStruct((n, 1), f32),
            jax.ShapeDtypeStruct((1, nhid), f32),
            jax.ShapeDtypeStruct((1, 1), f32),
        ],
    )(x, W_init, b_init.reshape(1, nhid), att_W, a1, a2)

    F2 = f2L.reshape(1, n)
    F2p = F2 * ALPHA

    mask_args = [hop_masks[k].view(jnp.uint8) for k in range(k_hops)]
    mask_specs = [pl.BlockSpec((bi, n), lambda i: (i, 0))
                  for _ in range(k_hops)]

    def attn(*refs):
        _attn_body(n, nhid, k_hops, *refs)

    out = pl.pallas_call(
        attn,
        grid=(n // bi,),
        in_specs=[
            pl.BlockSpec((1, 1), lambda i: (0, 0)),
            pl.BlockSpec((bi, 1), lambda i: (i, 0)),
            pl.BlockSpec((1, n), lambda i: (0, 0)),
            pl.BlockSpec((1, n), lambda i: (0, 0)),
            *mask_specs,
            pl.BlockSpec((n, nhid + 1), lambda i: (0, 0)),
            pl.BlockSpec((1, nhid), lambda i: (0, 0)),
            pl.BlockSpec((nhid, nhid), lambda i: (0, 0)),
            pl.BlockSpec((1, nhid), lambda i: (0, 0)),
            pl.BlockSpec((nhid, nclass), lambda i: (0, 0)),
            pl.BlockSpec((1, nclass), lambda i: (0, 0)),
        ],
        out_specs=pl.BlockSpec((bi, nclass), lambda i: (i, 0)),
        out_shape=jax.ShapeDtypeStruct((n, nclass), f32),
    )(m2, f1, F2, F2p, *mask_args, wha, whsum, appnp_W,
      appnp_b.reshape(1, nhid), W_out, b_out.reshape(1, nclass))
    return out
